# Initial kernel scaffold; baseline (speedup 1.0000x reference)
import jax
import jax.numpy as jnp
from jax.experimental import pallas as pl


def kernel(obj, caption, puzzle):
    raise NotImplementedError("write your pallas kernel here")


# speedup vs baseline: 3.5743x; 3.5743x over previous
import jax
import jax.numpy as jnp
from jax.experimental import pallas as pl

CLASSES = 10199
BATCH = 16384
ROWS_PER_BLOCK = 256


def _onehot_block(cap_ref, out_ref):
    cap = cap_ref[:, :]
    cols = jax.lax.broadcasted_iota(jnp.int32, (ROWS_PER_BLOCK, CLASSES), 1)
    out_ref[:, :] = (cols == cap).astype(jnp.float32)


def kernel(obj, caption, puzzle):
    cap = caption[0][:, None]
    grid = BATCH // ROWS_PER_BLOCK
    return pl.pallas_call(
        _onehot_block,
        grid=(grid,),
        in_specs=[pl.BlockSpec((ROWS_PER_BLOCK, 1), lambda i: (i, 0))],
        out_specs=pl.BlockSpec((ROWS_PER_BLOCK, CLASSES), lambda i: (i, 0)),
        out_shape=jax.ShapeDtypeStruct((BATCH, CLASSES), jnp.float32),
    )(cap)
